# Initial kernel scaffold; baseline (speedup 1.0000x reference)
import jax
import jax.numpy as jnp
from jax.experimental import pallas as pl


def kernel(x, iou_edge, sim_edge, W1, att_src1, att_dst1, bias1, W2, att_src2, att_dst2, bias2, Wn, bn, Wref, bref, Wbb, bbb):
    raise NotImplementedError("write your pallas kernel here")


# speedup vs baseline: 50.4807x; 50.4807x over previous
import functools

import numpy as np

import jax
import jax.numpy as jnp
from jax import lax
from jax.experimental import pallas as pl
from jax.experimental.pallas import tpu as pltpu
from jax.experimental.pallas import tpu_sc as plsc

NN = 10000
DD = 2048
EE = 640000
NP = 10112
RPS = NP // 16

B = 112
NW = 32
NB = (EE + NN + NW * B - 1) // (NW * B)
PERW = NB * B
EP = NW * PERW

TW = 128
W1L = 80
W2L = 32

_P80 = np.zeros((64, W1L), np.float32)
for _h in range(8):
    for _c in range(8):
        _P80[_h * 8 + _c, _h * 10 + _c] = 1.0
_S80 = np.zeros((W1L, W1L), np.float32)
for _c in range(W1L):
    _S80[(_c // 10) * 10 + 8, _c] = 1.0
_G80 = np.zeros((W1L, 64), np.float32)
for _h in range(8):
    for _c in range(8):
        _G80[_h * 10 + _c, _h * 8 + _c] = 1.0


def _tca_body(x_ref, w_ref, p1_ref, bias_ref, pre_ref, t1_ref, mx_ref):
    m_i = pl.program_id(0)
    k_i = pl.program_id(1)

    @pl.when(k_i == 0)
    def _init():
        pre_ref[...] = jnp.zeros_like(pre_ref)

    pre_ref[...] += jnp.dot(x_ref[...], w_ref[...],
                            preferred_element_type=jnp.float32)

    @pl.when(k_i == 3)
    def _fin():
        pre = pre_ref[...] + bias_ref[...]
        pre_ref[...] = pre
        h64 = pre[:, :64]
        col = lax.broadcasted_iota(jnp.int32, (400, TW), 1)
        ones80 = jnp.where((col % 10 == 8) & (col < W1L), 1.0, 0.0)
        t1 = jnp.dot(h64, p1_ref[...],
                     preferred_element_type=jnp.float32) + ones80
        t1_ref[...] = t1
        bs = jnp.max(t1[:, W1L:W1L + 16], axis=0, keepdims=True)
        bd = jnp.max(t1[:, W1L + 16:W1L + 32], axis=0, keepdims=True)
        upd = jnp.concatenate(
            [bs, bd, jnp.full((6, 16), -1e30, jnp.float32)], axis=0)

        @pl.when(m_i == 0)
        def _mx0():
            mx_ref[...] = upd

        @pl.when(m_i > 0)
        def _mxu():
            mx_ref[...] = jnp.maximum(mx_ref[...], upd)


def _run_tca(x, wcat, p1, bias_cat):
    return pl.pallas_call(
        _tca_body,
        grid=(25, 4),
        in_specs=[
            pl.BlockSpec((400, 512), lambda m, k: (m, k)),
            pl.BlockSpec((512, 384), lambda m, k: (k, 0)),
            pl.BlockSpec((64, TW), lambda m, k: (0, 0)),
            pl.BlockSpec((1, 384), lambda m, k: (0, 0)),
        ],
        out_specs=[
            pl.BlockSpec((400, 384), lambda m, k: (m, 0)),
            pl.BlockSpec((400, TW), lambda m, k: (m, 0)),
            pl.BlockSpec((8, 16), lambda m, k: (0, 0)),
        ],
        out_shape=[
            jax.ShapeDtypeStruct((NN, 384), jnp.float32),
            jax.ShapeDtypeStruct((NN, TW), jnp.float32),
            jax.ShapeDtypeStruct((8, 16), jnp.float32),
        ],
    )(x, wcat, p1, bias_cat)


def _make_sc_edge(accw, a_off, d_off, nsl, gather_ex):
    mesh = plsc.VectorSubcoreMesh(core_axis_name="c", subcore_axis_name="s")
    scratch = [
        pltpu.VMEM((B,), jnp.int32),
        pltpu.VMEM((B,), jnp.int32),
        pltpu.VMEM((B, TW), jnp.float32),
        pltpu.VMEM((B, TW), jnp.float32),
        pltpu.VMEM((16,), jnp.float32),
        pltpu.VMEM((B, TW), jnp.float32),
        pltpu.VMEM((8, 16), jnp.float32),
        pltpu.VMEM_SHARED((NP, TW), jnp.float32),
        pltpu.SemaphoreType.DMA,
        pltpu.SemaphoreType.DMA,
    ]

    @functools.partial(
        pl.kernel, mesh=mesh,
        out_type=jax.ShapeDtypeStruct((2, NP, TW), jnp.float32),
        scratch_types=scratch,
        compiler_params=pltpu.CompilerParams(needs_layout_passes=False),
    )
    def _sc(srcp, dstp, tab, mx, zer, acc_out,
            src_v, dst_v, s_v, d_v, exd_v, comb_v, mx_v,
            acc_sh, sem1, sem2):
        cid = lax.axis_index("c")
        sid = lax.axis_index("s")
        wid = sid * 2 + cid
        pltpu.sync_copy(zer.at[pl.ds(sid * RPS, RPS)],
                        acc_sh.at[pl.ds(sid * RPS, RPS)])
        pltpu.sync_copy(mx, mx_v)
        ksum = mx_v[0, :] + mx_v[1, :]
        k16 = jnp.where(ksum > 0.0, ksum, 0.2 * ksum)
        lane = lax.iota(jnp.int32, 16)
        hidx = [(lane + (j * 16)) // 10 for j in range(nsl)]
        zv = jnp.zeros((16,), jnp.float32)

        def zrow(b, zcarry):
            for j in range(nsl, TW // 16):
                comb_v[b, pl.ds(j * 16, 16)] = zv
            return zcarry

        lax.fori_loop(0, B, zrow, 0)
        plsc.subcore_barrier()
        base0 = wid * PERW

        def blk(i, carry):
            base = base0 + i * B
            pltpu.sync_copy(srcp.at[pl.ds(base, B)], src_v)
            pltpu.sync_copy(dstp.at[pl.ds(base, B)], dst_v)
            ca = pltpu.async_copy(tab.at[src_v], s_v, sem1)
            cb = pltpu.async_copy(tab.at[dst_v], d_v, sem2)
            ca.wait()
            cb.wait()

            def edge(b, ecarry):
                av = s_v[b, pl.ds(a_off, 16)]
                dv = d_v[b, pl.ds(d_off, 16)]
                s = av + dv
                ex = jnp.exp(jnp.where(s > 0.0, s, 0.2 * s) - k16)
                if gather_ex:
                    exd_v[...] = ex
                    for j in range(nsl):
                        hh = s_v[b, pl.ds(j * 16, 16)]
                        exg = plsc.load_gather(exd_v, [hidx[j]])
                        comb_v[b, pl.ds(j * 16, 16)] = exg * hh
                else:
                    for j in range(nsl):
                        hh = s_v[b, pl.ds(j * 16, 16)]
                        comb_v[b, pl.ds(j * 16, 16)] = ex * hh
                return ecarry

            lax.fori_loop(0, B, edge, 0)
            pltpu.sync_copy(comb_v, acc_sh.at[dst_v], add=True)
            return carry

        lax.fori_loop(0, NB, blk, 0)
        plsc.subcore_barrier()
        pltpu.sync_copy(acc_sh.at[pl.ds(sid * RPS, RPS)],
                        acc_out.at[cid, pl.ds(sid * RPS, RPS)])

    return _sc


_sc_edge1 = _make_sc_edge(W1L, 80, 96, 5, True)
_sc_edge2 = _make_sc_edge(W2L, 32, 48, 2, False)


def _tcb_body(a0_ref, a1_ref, s_ref, g_ref, b1_ref, w2_ref, as2_ref, ad2_ref,
              t2_ref, mx2_ref):
    m_i = pl.program_id(0)
    acc = a0_ref[...] + a1_ref[...]
    den = jnp.dot(acc, s_ref[...], preferred_element_type=jnp.float32)
    rat = acc / (den + 1e-16)
    out1 = jnp.dot(rat, g_ref[...],
                   preferred_element_type=jnp.float32) + b1_ref[...]
    h1 = jnp.where(out1 > 0.0, out1, jnp.exp(jnp.minimum(out1, 0.0)) - 1.0)
    h2p = jnp.dot(h1, w2_ref[...], preferred_element_type=jnp.float32)
    col = lax.broadcasted_iota(jnp.int32, h2p.shape, 1)
    h2x = h2p + jnp.where(col == 21, 1.0, 0.0)
    als2 = jnp.dot(h2p, as2_ref[...], preferred_element_type=jnp.float32)
    ald2 = jnp.dot(h2p, ad2_ref[...], preferred_element_type=jnp.float32)
    nrow = h2p.shape[0]
    t2_ref[...] = jnp.concatenate(
        [h2x, als2, ald2, jnp.zeros((nrow, TW - W2L - 32), jnp.float32)],
        axis=1)
    bs = jnp.max(als2, axis=0, keepdims=True)
    bd = jnp.max(ald2, axis=0, keepdims=True)
    upd = jnp.concatenate(
        [bs, bd, jnp.full((6, 16), -1e30, jnp.float32)], axis=0)

    @pl.when(m_i == 0)
    def _mx0():
        mx2_ref[...] = upd

    @pl.when(m_i > 0)
    def _mxu():
        mx2_ref[...] = jnp.maximum(mx2_ref[...], upd)


def _run_tcb(a0, a1, s80, g80, b1r, w2p, as2m, ad2m):
    mrows = NP // 4
    return pl.pallas_call(
        _tcb_body,
        grid=(4,),
        in_specs=[
            pl.BlockSpec((mrows, W1L), lambda m: (m, 0)),
            pl.BlockSpec((mrows, W1L), lambda m: (m, 0)),
            pl.BlockSpec((W1L, W1L), lambda m: (0, 0)),
            pl.BlockSpec((W1L, 64), lambda m: (0, 0)),
            pl.BlockSpec((1, 64), lambda m: (0, 0)),
            pl.BlockSpec((64, W2L), lambda m: (0, 0)),
            pl.BlockSpec((W2L, 16), lambda m: (0, 0)),
            pl.BlockSpec((W2L, 16), lambda m: (0, 0)),
        ],
        out_specs=[
            pl.BlockSpec((mrows, TW), lambda m: (m, 0)),
            pl.BlockSpec((8, 16), lambda m: (0, 0)),
        ],
        out_shape=[
            jax.ShapeDtypeStruct((NP, TW), jnp.float32),
            jax.ShapeDtypeStruct((8, 16), jnp.float32),
        ],
    )(a0, a1, s80, g80, b1r, w2p, as2m, ad2m)


def _tcc_body(a0_ref, a1_ref, b2_ref, wn_ref, bn_ref, ns_ref, gs_ref):
    m_i = pl.program_id(0)
    acc = a0_ref[...] + a1_ref[...]
    col = lax.broadcasted_iota(jnp.int32, acc.shape, 1)
    densel = jnp.where(col == 21, 1.0, 0.0)
    den = jnp.sum(acc * densel, axis=1, keepdims=True)
    h2 = acc / (den + 1e-16) + b2_ref[...]
    logits = jnp.dot(h2, wn_ref[...],
                     preferred_element_type=jnp.float32) + bn_ref[...]
    logits = logits + jnp.where(col >= 21, -1e30, 0.0)
    rowmax = jnp.max(logits, axis=1, keepdims=True)
    eo = jnp.exp(logits - rowmax)
    ns = eo / jnp.sum(eo, axis=1, keepdims=True)
    ns_ref[...] = ns
    rid = lax.broadcasted_iota(jnp.int32, acc.shape, 0) + m_i * acc.shape[0]
    gpart = jnp.sum(jnp.where(rid < NN, ns, 0.0), axis=0,
                    keepdims=True) * (1.0 / NN)

    @pl.when(m_i == 0)
    def _g0():
        gs_ref[...] = gpart

    @pl.when(m_i > 0)
    def _gu():
        gs_ref[...] += gpart


def _run_tcc(a0, a1, b2r, wnp, bnp):
    mrows = NP // 4
    return pl.pallas_call(
        _tcc_body,
        grid=(4,),
        in_specs=[
            pl.BlockSpec((mrows, W2L), lambda m: (m, 0)),
            pl.BlockSpec((mrows, W2L), lambda m: (m, 0)),
            pl.BlockSpec((1, W2L), lambda m: (0, 0)),
            pl.BlockSpec((W2L, W2L), lambda m: (0, 0)),
            pl.BlockSpec((1, W2L), lambda m: (0, 0)),
        ],
        out_specs=[
            pl.BlockSpec((mrows, W2L), lambda m: (m, 0)),
            pl.BlockSpec((1, W2L), lambda m: (0, 0)),
        ],
        out_shape=[
            jax.ShapeDtypeStruct((NP, W2L), jnp.float32),
            jax.ShapeDtypeStruct((1, W2L), jnp.float32),
        ],
    )(a0, a1, b2r, wnp, bnp)


def kernel(x, iou_edge, sim_edge, W1, att_src1, att_dst1, bias1,
           W2, att_src2, att_dst2, bias2, Wn, bn, Wref, bref, Wbb, bbb):
    f32 = jnp.float32
    loops = jnp.arange(NN, dtype=jnp.int32)
    pad = EP - (EE + NN)
    pad_src = jnp.zeros((pad,), jnp.int32)
    pad_dst = jnp.full((pad,), NN, jnp.int32)
    s1 = jnp.concatenate([iou_edge[0], loops, pad_src])
    d1 = jnp.concatenate([iou_edge[1], loops, pad_dst])
    s2 = jnp.concatenate([sim_edge[0], loops, pad_src])
    d2 = jnp.concatenate([sim_edge[1], loops, pad_dst])

    wcat = jnp.concatenate([W1, Wref, Wbb, jnp.zeros((DD, 5), f32)], axis=1)
    bias_cat = jnp.concatenate(
        [jnp.zeros((64,), f32), bref, bbb, jnp.zeros((5,), f32)]
    ).reshape(1, 384)
    eye8 = jnp.eye(8, dtype=f32)
    as1 = (att_src1[:, :, None] * eye8[:, None, :]).reshape(64, 8)
    ad1 = (att_dst1[:, :, None] * eye8[:, None, :]).reshape(64, 8)
    p1 = jnp.concatenate(
        [jnp.asarray(_P80), as1, as1, ad1, ad1, jnp.zeros((64, 16), f32)],
        axis=1)
    s80 = jnp.asarray(_S80)
    g80 = jnp.asarray(_G80)
    b1r = bias1.reshape(1, 64)
    w2p = jnp.concatenate([W2, jnp.zeros((64, W2L - 21), f32)], axis=1)
    a2s = jnp.concatenate([att_src2.reshape(-1), jnp.zeros((11,), f32)])
    a2d = jnp.concatenate([att_dst2.reshape(-1), jnp.zeros((11,), f32)])
    as2m = jnp.broadcast_to(a2s[:, None], (W2L, 16))
    ad2m = jnp.broadcast_to(a2d[:, None], (W2L, 16))
    wnp = jnp.zeros((W2L, W2L), f32).at[:21, :21].set(Wn)
    bnp = jnp.concatenate([bn, jnp.zeros((11,), f32)]).reshape(1, W2L)
    b2r = jnp.concatenate([bias2, jnp.zeros((11,), f32)]).reshape(1, W2L)
    z1 = jnp.zeros((NP, TW), f32)
    z2 = jnp.zeros((NP, TW), f32)

    pre, t1, mx1 = _run_tca(x, wcat, p1, bias_cat)
    t1p = jnp.concatenate([t1, jnp.zeros((NP - NN, TW), f32)], axis=0)

    acc1 = _sc_edge1(s1, d1, t1p, mx1, z1)

    t2, mx2 = _run_tcb(acc1[0, :, :W1L], acc1[1, :, :W1L], s80, g80, b1r, w2p, as2m, ad2m)

    acc2 = _sc_edge2(s2, d2, t2, mx2, z2)

    ns, gs = _run_tcc(acc2[0, :, :W2L], acc2[1, :, :W2L], b2r, wnp, bnp)

    node_score = ns[:NN, :21]
    graph_score = gs[0, :21]
    ref1 = pre[:, 64:85]
    ref2 = pre[:, 85:106]
    ref3 = pre[:, 106:127]
    bb1 = pre[:, 127:211]
    bb2 = pre[:, 211:295]
    bb3 = pre[:, 295:379]
    return (graph_score, node_score, ref1, ref2, ref3, bb1, bb2, bb3)


# speedup vs baseline: 51.2241x; 1.0147x over previous
import functools

import numpy as np

import jax
import jax.numpy as jnp
from jax import lax
from jax.experimental import pallas as pl
from jax.experimental.pallas import tpu as pltpu
from jax.experimental.pallas import tpu_sc as plsc

NN = 10000
DD = 2048
EE = 640000
NP = 10112
RPS = NP // 16

B = 112
NW = 32
NB = (EE + NN + NW * B - 1) // (NW * B)
PERW = NB * B
EP = NW * PERW

TW = 128
W1L = 80
W2L = 32

_P80 = np.zeros((64, W1L), np.float32)
for _h in range(8):
    for _c in range(8):
        _P80[_h * 8 + _c, _h * 10 + _c] = 1.0
_S80 = np.zeros((W1L, W1L), np.float32)
for _c in range(W1L):
    _S80[(_c // 10) * 10 + 8, _c] = 1.0
_G80 = np.zeros((W1L, 64), np.float32)
for _h in range(8):
    for _c in range(8):
        _G80[_h * 10 + _c, _h * 8 + _c] = 1.0


def _tca_body(x_ref, w_ref, p1_ref, bias_ref, pre_ref, t1_ref, mx_ref):
    m_i = pl.program_id(0)
    k_i = pl.program_id(1)

    @pl.when(k_i == 0)
    def _init():
        pre_ref[...] = jnp.zeros_like(pre_ref)

    pre_ref[...] += jnp.dot(x_ref[...], w_ref[...],
                            preferred_element_type=jnp.float32)

    @pl.when(k_i == 3)
    def _fin():
        pre = pre_ref[...] + bias_ref[...]
        pre_ref[...] = pre
        h64 = pre[:, :64]
        col = lax.broadcasted_iota(jnp.int32, (400, TW), 1)
        ones80 = jnp.where((col % 10 == 8) & (col < W1L), 1.0, 0.0)
        t1 = jnp.dot(h64, p1_ref[...],
                     preferred_element_type=jnp.float32) + ones80
        t1_ref[...] = t1
        bs = jnp.max(t1[:, W1L:W1L + 16], axis=0, keepdims=True)
        bd = jnp.max(t1[:, W1L + 16:W1L + 32], axis=0, keepdims=True)
        upd = jnp.concatenate(
            [bs, bd, jnp.full((6, 16), -1e30, jnp.float32)], axis=0)

        @pl.when(m_i == 0)
        def _mx0():
            mx_ref[...] = upd

        @pl.when(m_i > 0)
        def _mxu():
            mx_ref[...] = jnp.maximum(mx_ref[...], upd)


def _run_tca(x, wcat, p1, bias_cat):
    return pl.pallas_call(
        _tca_body,
        grid=(25, 4),
        in_specs=[
            pl.BlockSpec((400, 512), lambda m, k: (m, k)),
            pl.BlockSpec((512, 384), lambda m, k: (k, 0)),
            pl.BlockSpec((64, TW), lambda m, k: (0, 0)),
            pl.BlockSpec((1, 384), lambda m, k: (0, 0)),
        ],
        out_specs=[
            pl.BlockSpec((400, 384), lambda m, k: (m, 0)),
            pl.BlockSpec((400, TW), lambda m, k: (m, 0)),
            pl.BlockSpec((8, 16), lambda m, k: (0, 0)),
        ],
        out_shape=[
            jax.ShapeDtypeStruct((NN, 384), jnp.float32),
            jax.ShapeDtypeStruct((NN, TW), jnp.float32),
            jax.ShapeDtypeStruct((8, 16), jnp.float32),
        ],
    )(x, wcat, p1, bias_cat)


def _make_sc_edge(accw, a_off, d_off, nsl, gather_ex):
    mesh = plsc.VectorSubcoreMesh(core_axis_name="c", subcore_axis_name="s")
    scratch = [
        pltpu.VMEM((B,), jnp.int32),
        pltpu.VMEM((B,), jnp.int32),
        pltpu.VMEM((B, TW), jnp.float32),
        pltpu.VMEM((B, TW), jnp.float32),
        pltpu.VMEM((32,), jnp.float32),
        pltpu.VMEM((B, TW), jnp.float32),
        pltpu.VMEM((8, 16), jnp.float32),
        pltpu.VMEM_SHARED((NP, TW), jnp.float32),
        pltpu.SemaphoreType.DMA,
        pltpu.SemaphoreType.DMA,
    ]

    @functools.partial(
        pl.kernel, mesh=mesh,
        out_type=jax.ShapeDtypeStruct((2, NP, TW), jnp.float32),
        scratch_types=scratch,
        compiler_params=pltpu.CompilerParams(needs_layout_passes=False),
    )
    def _sc(srcp, dstp, tab, mx, zer, acc_out,
            src_v, dst_v, s_v, d_v, exd_v, comb_v, mx_v,
            acc_sh, sem1, sem2):
        cid = lax.axis_index("c")
        sid = lax.axis_index("s")
        wid = sid * 2 + cid
        pltpu.sync_copy(zer.at[pl.ds(sid * RPS, RPS)],
                        acc_sh.at[pl.ds(sid * RPS, RPS)])
        pltpu.sync_copy(mx, mx_v)
        ksum = mx_v[0, :] + mx_v[1, :]
        k16 = jnp.where(ksum > 0.0, ksum, 0.2 * ksum)
        lane = lax.iota(jnp.int32, 16)
        hidx = [(lane + (j * 16)) // 10 for j in range(nsl)]
        zv = jnp.zeros((16,), jnp.float32)

        def zrow(b, zcarry):
            for j in range(nsl, TW // 16):
                comb_v[b, pl.ds(j * 16, 16)] = zv
            return zcarry

        lax.fori_loop(0, B, zrow, 0)
        plsc.subcore_barrier()
        base0 = wid * PERW

        def blk(i, carry):
            base = base0 + i * B
            pltpu.sync_copy(srcp.at[pl.ds(base, B)], src_v)
            pltpu.sync_copy(dstp.at[pl.ds(base, B)], dst_v)
            ca = pltpu.async_copy(tab.at[src_v], s_v, sem1)
            cb = pltpu.async_copy(tab.at[dst_v], d_v, sem2)
            ca.wait()
            cb.wait()

            def edge(bp, ecarry):
                for u in range(2):
                    b = bp * 2 + u
                    av = s_v[b, pl.ds(a_off, 16)]
                    dv = d_v[b, pl.ds(d_off, 16)]
                    s = av + dv
                    ex = jnp.exp(jnp.where(s > 0.0, s, 0.2 * s) - k16)
                    if gather_ex:
                        exd_v[pl.ds(u * 16, 16)] = ex
                        for j in range(nsl):
                            hh = s_v[b, pl.ds(j * 16, 16)]
                            exg = plsc.load_gather(exd_v,
                                                   [u * 16 + hidx[j]])
                            comb_v[b, pl.ds(j * 16, 16)] = exg * hh
                    else:
                        for j in range(nsl):
                            hh = s_v[b, pl.ds(j * 16, 16)]
                            comb_v[b, pl.ds(j * 16, 16)] = ex * hh
                return ecarry

            lax.fori_loop(0, B // 2, edge, 0)
            pltpu.sync_copy(comb_v, acc_sh.at[dst_v], add=True)
            return carry

        lax.fori_loop(0, NB, blk, 0)
        plsc.subcore_barrier()
        pltpu.sync_copy(acc_sh.at[pl.ds(sid * RPS, RPS)],
                        acc_out.at[cid, pl.ds(sid * RPS, RPS)])

    return _sc


_sc_edge1 = _make_sc_edge(W1L, 80, 96, 5, True)
_sc_edge2 = _make_sc_edge(W2L, 32, 48, 2, False)


def _tcb_body(a0_ref, a1_ref, s_ref, g_ref, b1_ref, w2_ref, as2_ref, ad2_ref,
              t2_ref, mx2_ref):
    m_i = pl.program_id(0)
    acc = a0_ref[...] + a1_ref[...]
    den = jnp.dot(acc, s_ref[...], preferred_element_type=jnp.float32)
    rat = acc / (den + 1e-16)
    out1 = jnp.dot(rat, g_ref[...],
                   preferred_element_type=jnp.float32) + b1_ref[...]
    h1 = jnp.where(out1 > 0.0, out1, jnp.exp(jnp.minimum(out1, 0.0)) - 1.0)
    h2p = jnp.dot(h1, w2_ref[...], preferred_element_type=jnp.float32)
    col = lax.broadcasted_iota(jnp.int32, h2p.shape, 1)
    h2x = h2p + jnp.where(col == 21, 1.0, 0.0)
    als2 = jnp.dot(h2p, as2_ref[...], preferred_element_type=jnp.float32)
    ald2 = jnp.dot(h2p, ad2_ref[...], preferred_element_type=jnp.float32)
    nrow = h2p.shape[0]
    t2_ref[...] = jnp.concatenate(
        [h2x, als2, ald2, jnp.zeros((nrow, TW - W2L - 32), jnp.float32)],
        axis=1)
    bs = jnp.max(als2, axis=0, keepdims=True)
    bd = jnp.max(ald2, axis=0, keepdims=True)
    upd = jnp.concatenate(
        [bs, bd, jnp.full((6, 16), -1e30, jnp.float32)], axis=0)

    @pl.when(m_i == 0)
    def _mx0():
        mx2_ref[...] = upd

    @pl.when(m_i > 0)
    def _mxu():
        mx2_ref[...] = jnp.maximum(mx2_ref[...], upd)


def _run_tcb(a0, a1, s80, g80, b1r, w2p, as2m, ad2m):
    mrows = NP // 4
    return pl.pallas_call(
        _tcb_body,
        grid=(4,),
        in_specs=[
            pl.BlockSpec((mrows, W1L), lambda m: (m, 0)),
            pl.BlockSpec((mrows, W1L), lambda m: (m, 0)),
            pl.BlockSpec((W1L, W1L), lambda m: (0, 0)),
            pl.BlockSpec((W1L, 64), lambda m: (0, 0)),
            pl.BlockSpec((1, 64), lambda m: (0, 0)),
            pl.BlockSpec((64, W2L), lambda m: (0, 0)),
            pl.BlockSpec((W2L, 16), lambda m: (0, 0)),
            pl.BlockSpec((W2L, 16), lambda m: (0, 0)),
        ],
        out_specs=[
            pl.BlockSpec((mrows, TW), lambda m: (m, 0)),
            pl.BlockSpec((8, 16), lambda m: (0, 0)),
        ],
        out_shape=[
            jax.ShapeDtypeStruct((NP, TW), jnp.float32),
            jax.ShapeDtypeStruct((8, 16), jnp.float32),
        ],
    )(a0, a1, s80, g80, b1r, w2p, as2m, ad2m)


def _tcc_body(a0_ref, a1_ref, b2_ref, wn_ref, bn_ref, ns_ref, gs_ref):
    m_i = pl.program_id(0)
    acc = a0_ref[...] + a1_ref[...]
    col = lax.broadcasted_iota(jnp.int32, acc.shape, 1)
    densel = jnp.where(col == 21, 1.0, 0.0)
    den = jnp.sum(acc * densel, axis=1, keepdims=True)
    h2 = acc / (den + 1e-16) + b2_ref[...]
    logits = jnp.dot(h2, wn_ref[...],
                     preferred_element_type=jnp.float32) + bn_ref[...]
    logits = logits + jnp.where(col >= 21, -1e30, 0.0)
    rowmax = jnp.max(logits, axis=1, keepdims=True)
    eo = jnp.exp(logits - rowmax)
    ns = eo / jnp.sum(eo, axis=1, keepdims=True)
    ns_ref[...] = ns
    rid = lax.broadcasted_iota(jnp.int32, acc.shape, 0) + m_i * acc.shape[0]
    gpart = jnp.sum(jnp.where(rid < NN, ns, 0.0), axis=0,
                    keepdims=True) * (1.0 / NN)

    @pl.when(m_i == 0)
    def _g0():
        gs_ref[...] = gpart

    @pl.when(m_i > 0)
    def _gu():
        gs_ref[...] += gpart


def _run_tcc(a0, a1, b2r, wnp, bnp):
    mrows = NP // 4
    return pl.pallas_call(
        _tcc_body,
        grid=(4,),
        in_specs=[
            pl.BlockSpec((mrows, W2L), lambda m: (m, 0)),
            pl.BlockSpec((mrows, W2L), lambda m: (m, 0)),
            pl.BlockSpec((1, W2L), lambda m: (0, 0)),
            pl.BlockSpec((W2L, W2L), lambda m: (0, 0)),
            pl.BlockSpec((1, W2L), lambda m: (0, 0)),
        ],
        out_specs=[
            pl.BlockSpec((mrows, W2L), lambda m: (m, 0)),
            pl.BlockSpec((1, W2L), lambda m: (0, 0)),
        ],
        out_shape=[
            jax.ShapeDtypeStruct((NP, W2L), jnp.float32),
            jax.ShapeDtypeStruct((1, W2L), jnp.float32),
        ],
    )(a0, a1, b2r, wnp, bnp)


def kernel(x, iou_edge, sim_edge, W1, att_src1, att_dst1, bias1,
           W2, att_src2, att_dst2, bias2, Wn, bn, Wref, bref, Wbb, bbb):
    f32 = jnp.float32
    loops = jnp.arange(NN, dtype=jnp.int32)
    pad = EP - (EE + NN)
    pad_src = jnp.zeros((pad,), jnp.int32)
    pad_dst = jnp.full((pad,), NN, jnp.int32)
    s1 = jnp.concatenate([iou_edge[0], loops, pad_src])
    d1 = jnp.concatenate([iou_edge[1], loops, pad_dst])
    s2 = jnp.concatenate([sim_edge[0], loops, pad_src])
    d2 = jnp.concatenate([sim_edge[1], loops, pad_dst])

    wcat = jnp.concatenate([W1, Wref, Wbb, jnp.zeros((DD, 5), f32)], axis=1)
    bias_cat = jnp.concatenate(
        [jnp.zeros((64,), f32), bref, bbb, jnp.zeros((5,), f32)]
    ).reshape(1, 384)
    eye8 = jnp.eye(8, dtype=f32)
    as1 = (att_src1[:, :, None] * eye8[:, None, :]).reshape(64, 8)
    ad1 = (att_dst1[:, :, None] * eye8[:, None, :]).reshape(64, 8)
    p1 = jnp.concatenate(
        [jnp.asarray(_P80), as1, as1, ad1, ad1, jnp.zeros((64, 16), f32)],
        axis=1)
    s80 = jnp.asarray(_S80)
    g80 = jnp.asarray(_G80)
    b1r = bias1.reshape(1, 64)
    w2p = jnp.concatenate([W2, jnp.zeros((64, W2L - 21), f32)], axis=1)
    a2s = jnp.concatenate([att_src2.reshape(-1), jnp.zeros((11,), f32)])
    a2d = jnp.concatenate([att_dst2.reshape(-1), jnp.zeros((11,), f32)])
    as2m = jnp.broadcast_to(a2s[:, None], (W2L, 16))
    ad2m = jnp.broadcast_to(a2d[:, None], (W2L, 16))
    wnp = jnp.zeros((W2L, W2L), f32).at[:21, :21].set(Wn)
    bnp = jnp.concatenate([bn, jnp.zeros((11,), f32)]).reshape(1, W2L)
    b2r = jnp.concatenate([bias2, jnp.zeros((11,), f32)]).reshape(1, W2L)
    z1 = jnp.zeros((NP, TW), f32)
    z2 = jnp.zeros((NP, TW), f32)

    pre, t1, mx1 = _run_tca(x, wcat, p1, bias_cat)
    t1p = jnp.concatenate([t1, jnp.zeros((NP - NN, TW), f32)], axis=0)

    acc1 = _sc_edge1(s1, d1, t1p, mx1, z1)

    t2, mx2 = _run_tcb(acc1[0, :, :W1L], acc1[1, :, :W1L], s80, g80, b1r, w2p, as2m, ad2m)

    acc2 = _sc_edge2(s2, d2, t2, mx2, z2)

    ns, gs = _run_tcc(acc2[0, :, :W2L], acc2[1, :, :W2L], b2r, wnp, bnp)

    node_score = ns[:NN, :21]
    graph_score = gs[0, :21]
    ref1 = pre[:, 64:85]
    ref2 = pre[:, 85:106]
    ref3 = pre[:, 106:127]
    bb1 = pre[:, 127:211]
    bb2 = pre[:, 211:295]
    bb3 = pre[:, 295:379]
    return (graph_score, node_score, ref1, ref2, ref3, bb1, bb2, bb3)
